# Initial kernel scaffold; baseline (speedup 1.0000x reference)
import jax
import jax.numpy as jnp
from jax.experimental import pallas as pl


def kernel(x, edge_index, edge_attr, Wg, bg, Wc, bc):
    raise NotImplementedError("write your pallas kernel here")


# speedup vs baseline: 1.3798x; 1.3798x over previous
import functools

import jax
import jax.numpy as jnp
from jax import lax
from jax.experimental import pallas as pl
from jax.experimental.pallas import tpu as pltpu
from jax.experimental.pallas import tpu_sc as plsc

N_NODES = 10000
N_EDGES = 160000
CH = 128
E_BLK = 80
NS = 16
EDGES_PER_TILE = N_EDGES // NS
CHUNKS = EDGES_PER_TILE // E_BLK
ACC_ROWS = 10240


def _gates_body(attr_ref, wT_ref, b_ref, out_ref):
    z = jnp.dot(attr_ref[...], wT_ref[0], preferred_element_type=jnp.float32)
    out_ref[...] = jax.nn.sigmoid(z + b_ref[0])


def _gates_tc(edge_attr, WgT_s, bg_s):
    BM = 640
    nm = N_EDGES // BM
    return pl.pallas_call(
        _gates_body,
        grid=(2, nm),
        in_specs=[
            pl.BlockSpec((BM, 16), lambda h, m: (m, 0)),
            pl.BlockSpec((1, 16, CH), lambda h, m: (h, 0, 0)),
            pl.BlockSpec((1, 1, CH), lambda h, m: (h, 0, 0)),
        ],
        out_specs=pl.BlockSpec((BM, CH), lambda h, m: (h * nm + m, 0)),
        out_shape=jax.ShapeDtypeStruct((2 * N_EDGES, CH), jnp.float32),
    )(edge_attr, WgT_s, bg_s)


def _sc_agg_body(src2_hbm, dst_hbm, x2_hbm, gates_hbm, out_hbm,
                 sidx_v, didx_v, rows_v, g_v, acc, sem):
    c = lax.axis_index("c")
    s = lax.axis_index("s")

    def zb(e, _):
        for j in range(CH // 16):
            rows_v[e, pl.ds(j * 16, 16)] = jnp.zeros((16,), jnp.float32)
        return 0
    lax.fori_loop(0, E_BLK, zb, 0)

    def zacc(k, _):
        pltpu.sync_copy(rows_v, acc.at[pl.ds(s * 640 + k * E_BLK, E_BLK)])
        return 0
    lax.fori_loop(0, ACC_ROWS // NS // E_BLK, zacc, 0)
    plsc.subcore_barrier()

    def step(k, _):
        base = s * EDGES_PER_TILE + k * E_BLK
        gbase = c * N_EDGES + base
        pltpu.sync_copy(src2_hbm.at[pl.ds(gbase, E_BLK)], sidx_v)
        pltpu.sync_copy(dst_hbm.at[pl.ds(base, E_BLK)], didx_v)
        pltpu.async_copy(x2_hbm.at[sidx_v], rows_v, sem).wait()
        pltpu.sync_copy(gates_hbm.at[pl.ds(gbase, E_BLK)], g_v)

        def mul(e, _):
            for j in range(CH // 16):
                sl = pl.ds(j * 16, 16)
                rows_v[e, sl] = rows_v[e, sl] * g_v[e, sl]
            return 0
        lax.fori_loop(0, E_BLK, mul, 0)

        pltpu.sync_copy(rows_v, acc.at[didx_v], add=True)
        return 0
    lax.fori_loop(0, CHUNKS, step, 0)
    plsc.subcore_barrier()

    @pl.when(s < NS - 1)
    def _():
        off = s * 624
        pltpu.sync_copy(acc.at[pl.ds(off, 624)],
                        out_hbm.at[pl.ds(c * N_NODES + off, 624)])

    @pl.when(s == NS - 1)
    def _():
        pltpu.sync_copy(acc.at[pl.ds(9360, 640)],
                        out_hbm.at[pl.ds(c * N_NODES + 9360, 640)])


def _sc_agg(src2, dst, x2, gates):
    mesh = plsc.VectorSubcoreMesh(core_axis_name="c", subcore_axis_name="s")
    fn = functools.partial(
        pl.kernel,
        mesh=mesh,
        out_type=jax.ShapeDtypeStruct((2 * N_NODES, CH), jnp.float32),
        scratch_types=[
            pltpu.VMEM((E_BLK,), jnp.int32),
            pltpu.VMEM((E_BLK,), jnp.int32),
            pltpu.VMEM((E_BLK, CH), jnp.float32),
            pltpu.VMEM((E_BLK, CH), jnp.float32),
            pltpu.VMEM_SHARED((ACC_ROWS, CH), jnp.float32),
            pltpu.SemaphoreType.DMA,
        ],
    )(_sc_agg_body)
    return fn(src2, dst, x2, gates)


def _combine_body(x_ref, a0_ref, a1_ref, w1_ref, w2a_ref, w2b_ref, b_ref, out_ref):
    acc = jnp.dot(x_ref[...], w1_ref[...], preferred_element_type=jnp.float32)
    acc += jnp.dot(a0_ref[...], w2a_ref[...], preferred_element_type=jnp.float32)
    acc += jnp.dot(a1_ref[...], w2b_ref[...], preferred_element_type=jnp.float32)
    out_ref[...] = acc + b_ref[...]


def _combine_tc(x, agg2, W1t, W2at, W2bt, bc2):
    BM = 1000
    nb = N_NODES // BM
    return pl.pallas_call(
        _combine_body,
        grid=(nb,),
        in_specs=[
            pl.BlockSpec((BM, 256), lambda m: (m, 0)),
            pl.BlockSpec((BM, CH), lambda m: (m, 0)),
            pl.BlockSpec((BM, CH), lambda m: (m + nb, 0)),
            pl.BlockSpec((256, 256), lambda m: (0, 0)),
            pl.BlockSpec((CH, 256), lambda m: (0, 0)),
            pl.BlockSpec((CH, 256), lambda m: (0, 0)),
            pl.BlockSpec((1, 256), lambda m: (0, 0)),
        ],
        out_specs=pl.BlockSpec((BM, 256), lambda m: (m, 0)),
        out_shape=jax.ShapeDtypeStruct((N_NODES, 256), jnp.float32),
    )(x, agg2, agg2, W1t, W2at, W2bt, bc2)


def kernel(x, edge_index, edge_attr, Wg, bg, Wc, bc):
    src = edge_index[0].astype(jnp.int32)
    dst = edge_index[1].astype(jnp.int32)
    src2 = jnp.concatenate([src, src + N_NODES])
    x2 = jnp.concatenate([x[:, :CH], x[:, CH:]], axis=0)

    WgT_s = jnp.stack([Wg[:CH].T, Wg[CH:].T])
    bg_s = jnp.stack([bg[:CH], bg[CH:]]).reshape(2, 1, CH)
    gates = _gates_tc(edge_attr, WgT_s, bg_s)

    agg2 = _sc_agg(src2, dst, x2, gates)

    WcT = Wc.T
    return _combine_tc(x, agg2, WcT[:256], WcT[256:256 + CH], WcT[256 + CH:],
                       bc.reshape(1, 256))


# speedup vs baseline: 3.0262x; 2.1932x over previous
import functools

import jax
import jax.numpy as jnp
from jax import lax
from jax.experimental import pallas as pl
from jax.experimental.pallas import tpu as pltpu
from jax.experimental.pallas import tpu_sc as plsc

N_NODES = 10000
N_EDGES = 160000
CH = 128
E_BLK = 80
NS = 16
EDGES_PER_TILE = N_EDGES // NS
CHUNKS = EDGES_PER_TILE // E_BLK
ACC_ROWS = 10240


def _gates_body(attr_ref, wT_ref, b_ref, out_ref):
    z = jnp.dot(attr_ref[...], wT_ref[0], preferred_element_type=jnp.float32)
    out_ref[...] = jax.nn.sigmoid(z + b_ref[0])


def _gates_tc(edge_attr, WgT_s, bg_s):
    BM = 4000
    nm = N_EDGES // BM
    return pl.pallas_call(
        _gates_body,
        grid=(2, nm),
        in_specs=[
            pl.BlockSpec((BM, 16), lambda h, m: (m, 0)),
            pl.BlockSpec((1, 16, CH), lambda h, m: (h, 0, 0)),
            pl.BlockSpec((1, 1, CH), lambda h, m: (h, 0, 0)),
        ],
        out_specs=pl.BlockSpec((BM, CH), lambda h, m: (h * nm + m, 0)),
        out_shape=jax.ShapeDtypeStruct((2 * N_EDGES, CH), jnp.float32),
        compiler_params=pltpu.CompilerParams(
            dimension_semantics=("parallel", "arbitrary")),
    )(edge_attr, WgT_s, bg_s)


def _sc_agg_body(src4_hbm, dst3_hbm, x2_hbm, gates_hbm, out_hbm,
                 idx_v, rows0, rows1, g0, g1, acc,
                 lsem0, lsem1, ssem0, ssem1, isem0, isem1, isem2, isem3):
    c = lax.axis_index("c")
    s = lax.axis_index("s")
    rows = (rows0, rows1)
    g = (g0, g1)
    lsem = (lsem0, lsem1)
    ssem = (ssem0, ssem1)
    isem = (isem0, isem1, isem2, isem3)

    def idx_load(k, slot):
        pltpu.async_copy(src4_hbm.at[c, s, k], idx_v.at[slot], isem[slot])
        pltpu.async_copy(dst3_hbm.at[s, k], idx_v.at[4 + slot], isem[slot])

    def idx_wait(k, slot):
        pltpu.make_async_copy(src4_hbm.at[c, s, k], idx_v.at[slot],
                              isem[slot]).wait()
        pltpu.make_async_copy(dst3_hbm.at[s, k], idx_v.at[4 + slot],
                              isem[slot]).wait()

    def gbase(k):
        return c * N_EDGES + s * EDGES_PER_TILE + k * E_BLK

    def load_start(k, slot, b):
        pltpu.async_copy(x2_hbm.at[idx_v.at[slot]], rows[b], lsem[b])
        pltpu.async_copy(gates_hbm.at[pl.ds(gbase(k), E_BLK)], g[b], lsem[b])

    def load_wait(k, slot, b):
        pltpu.make_async_copy(x2_hbm.at[idx_v.at[slot]], rows[b],
                              lsem[b]).wait()
        pltpu.make_async_copy(gates_hbm.at[pl.ds(gbase(k), E_BLK)], g[b],
                              lsem[b]).wait()

    def scat_start(slot, b):
        pltpu.async_copy(rows[b], acc.at[idx_v.at[4 + slot]], ssem[b],
                         add=True)

    def scat_wait(slot, b):
        pltpu.make_async_copy(rows[b], acc.at[idx_v.at[4 + slot]],
                              ssem[b]).wait()

    def multiply(b):
        rv, gv = rows[b], g[b]

        @plsc.parallel_loop(0, E_BLK, unroll=2)
        def _(e):
            for j in range(CH // 16):
                sl = pl.ds(j * 16, 16)
                rv[e, sl] = rv[e, sl] * gv[e, sl]

    @plsc.parallel_loop(0, E_BLK, unroll=2)
    def _(e):
        for j in range(CH // 16):
            rows0[e, pl.ds(j * 16, 16)] = jnp.zeros((16,), jnp.float32)

    def zacc(k, _):
        pltpu.sync_copy(rows0, acc.at[pl.ds(s * 640 + k * E_BLK, E_BLK)])
        return 0
    lax.fori_loop(0, ACC_ROWS // NS // E_BLK, zacc, 0)
    plsc.subcore_barrier()

    pltpu.sync_copy(src4_hbm.at[c, s, 0], idx_v.at[0])
    pltpu.sync_copy(dst3_hbm.at[s, 0], idx_v.at[4])
    idx_load(1, 1)
    load_start(0, 0, 0)

    def substep(k, q, b4):
        b = b4 % 2
        load_wait(k, b4, b)

        @pl.when(k > 0)
        def _():
            scat_wait((b4 + 3) % 4, 1 - b)

        @pl.when(k < CHUNKS - 2)
        def _():
            idx_load(k + 2, (b4 + 2) % 4)

        idx_wait(k + 1, (b4 + 1) % 4)
        load_start(k + 1, (b4 + 1) % 4, 1 - b)
        multiply(b)
        scat_start(b4, b)

    def quad(q, _):
        for b4 in range(4):
            substep(4 * q + b4, q, b4)
        return 0
    lax.fori_loop(0, (CHUNKS - 1) // 4, quad, 0)

    kt = CHUNKS - 1
    load_wait(kt, kt % 4, kt % 2)
    scat_wait((kt + 3) % 4, 1 - kt % 2)
    multiply(kt % 2)
    scat_start(kt % 4, kt % 2)
    scat_wait(kt % 4, kt % 2)
    plsc.subcore_barrier()

    @pl.when(s < NS - 1)
    def _():
        off = s * 624
        pltpu.sync_copy(acc.at[pl.ds(off, 624)],
                        out_hbm.at[pl.ds(c * N_NODES + off, 624)])

    @pl.when(s == NS - 1)
    def _():
        pltpu.sync_copy(acc.at[pl.ds(9360, 640)],
                        out_hbm.at[pl.ds(c * N_NODES + 9360, 640)])


def _sc_agg(src4, dst3, x2, gates):
    mesh = plsc.VectorSubcoreMesh(core_axis_name="c", subcore_axis_name="s")
    fn = functools.partial(
        pl.kernel,
        mesh=mesh,
        out_type=jax.ShapeDtypeStruct((2 * N_NODES, CH), jnp.float32),
        scratch_types=[
            pltpu.VMEM((8, E_BLK), jnp.int32),
            pltpu.VMEM((E_BLK, CH), jnp.float32),
            pltpu.VMEM((E_BLK, CH), jnp.float32),
            pltpu.VMEM((E_BLK, CH), jnp.float32),
            pltpu.VMEM((E_BLK, CH), jnp.float32),
            pltpu.VMEM_SHARED((ACC_ROWS, CH), jnp.float32),
        ] + [pltpu.SemaphoreType.DMA] * 8,
    )(_sc_agg_body)
    return fn(src4, dst3, x2, gates)


def _combine_body(x_ref, a0_ref, a1_ref, w1_ref, w2a_ref, w2b_ref, b_ref, out_ref):
    acc = jnp.dot(x_ref[...], w1_ref[...], preferred_element_type=jnp.float32)
    acc += jnp.dot(a0_ref[...], w2a_ref[...], preferred_element_type=jnp.float32)
    acc += jnp.dot(a1_ref[...], w2b_ref[...], preferred_element_type=jnp.float32)
    out_ref[...] = acc + b_ref[...]


def _combine_tc(x, agg2, W1t, W2at, W2bt, bc2):
    BM = 1000
    nb = N_NODES // BM
    return pl.pallas_call(
        _combine_body,
        grid=(nb,),
        in_specs=[
            pl.BlockSpec((BM, 256), lambda m: (m, 0)),
            pl.BlockSpec((BM, CH), lambda m: (m, 0)),
            pl.BlockSpec((BM, CH), lambda m: (m + nb, 0)),
            pl.BlockSpec((256, 256), lambda m: (0, 0)),
            pl.BlockSpec((CH, 256), lambda m: (0, 0)),
            pl.BlockSpec((CH, 256), lambda m: (0, 0)),
            pl.BlockSpec((1, 256), lambda m: (0, 0)),
        ],
        out_specs=pl.BlockSpec((BM, 256), lambda m: (m, 0)),
        out_shape=jax.ShapeDtypeStruct((N_NODES, 256), jnp.float32),
    )(x, agg2, agg2, W1t, W2at, W2bt, bc2)


def kernel(x, edge_index, edge_attr, Wg, bg, Wc, bc):
    src = edge_index[0].astype(jnp.int32)
    dst = edge_index[1].astype(jnp.int32)
    src4 = jnp.stack([src, src + N_NODES]).reshape(2, NS, CHUNKS, E_BLK)
    dst3 = dst.reshape(NS, CHUNKS, E_BLK)
    x2 = jnp.concatenate([x[:, :CH], x[:, CH:]], axis=0)

    WgT_s = jnp.stack([Wg[:CH].T, Wg[CH:].T])
    bg_s = jnp.stack([bg[:CH], bg[CH:]]).reshape(2, 1, CH)
    gates = _gates_tc(edge_attr, WgT_s, bg_s)

    agg2 = _sc_agg(src4, dst3, x2, gates)

    WcT = Wc.T
    return _combine_tc(x, agg2, WcT[:256], WcT[256:256 + CH], WcT[256 + CH:],
                       bc.reshape(1, 256))


# speedup vs baseline: 3.0751x; 1.0161x over previous
import functools

import jax
import jax.numpy as jnp
from jax import lax
from jax.experimental import pallas as pl
from jax.experimental.pallas import tpu as pltpu
from jax.experimental.pallas import tpu_sc as plsc

N_NODES = 10000
N_EDGES = 160000
CH = 128
E_BLK = 80
NS = 16
EDGES_PER_TILE = N_EDGES // NS
CHUNKS = EDGES_PER_TILE // E_BLK
ACC_ROWS = 10240


def _gates_body(attr_ref, wT_ref, b_ref, out_ref):
    z = jnp.dot(attr_ref[...], wT_ref[0], preferred_element_type=jnp.float32)
    out_ref[...] = jax.nn.sigmoid(z + b_ref[0])


def _gates_tc(edge_attr, WgT_s, bg_s):
    BM = 4000
    nm = N_EDGES // BM
    return pl.pallas_call(
        _gates_body,
        grid=(2, nm),
        in_specs=[
            pl.BlockSpec((BM, 16), lambda h, m: (m, 0)),
            pl.BlockSpec((1, 16, CH), lambda h, m: (h, 0, 0)),
            pl.BlockSpec((1, 1, CH), lambda h, m: (h, 0, 0)),
        ],
        out_specs=pl.BlockSpec((BM, CH), lambda h, m: (h * nm + m, 0)),
        out_shape=jax.ShapeDtypeStruct((2 * N_EDGES, CH), jnp.float32),
        compiler_params=pltpu.CompilerParams(
            dimension_semantics=("parallel", "arbitrary")),
    )(edge_attr, WgT_s, bg_s)


def _sc_agg_body(src4_hbm, dst3_hbm, x2_hbm, gates_hbm, out_hbm,
                 idx_v, rows0, rows1, g0, g1, acc,
                 lsem0, lsem1, ssem0, ssem1, isem0, isem1, isem2, isem3):
    c = lax.axis_index("c")
    s = lax.axis_index("s")
    rows = (rows0, rows1)
    g = (g0, g1)
    lsem = (lsem0, lsem1)
    ssem = (ssem0, ssem1)
    isem = (isem0, isem1, isem2, isem3)

    def idx_load(k, slot):
        pltpu.async_copy(src4_hbm.at[c, s, k], idx_v.at[slot], isem[slot])
        pltpu.async_copy(dst3_hbm.at[s, k], idx_v.at[4 + slot], isem[slot])

    def idx_wait(k, slot):
        pltpu.make_async_copy(src4_hbm.at[c, s, k], idx_v.at[slot],
                              isem[slot]).wait()
        pltpu.make_async_copy(dst3_hbm.at[s, k], idx_v.at[4 + slot],
                              isem[slot]).wait()

    def gbase(k):
        return c * N_EDGES + s * EDGES_PER_TILE + k * E_BLK

    def load_start(k, slot, b):
        pltpu.async_copy(x2_hbm.at[idx_v.at[slot]], rows[b], lsem[b])
        pltpu.async_copy(gates_hbm.at[pl.ds(gbase(k), E_BLK)], g[b], lsem[b])

    def load_wait(k, slot, b):
        pltpu.make_async_copy(x2_hbm.at[idx_v.at[slot]], rows[b],
                              lsem[b]).wait()
        pltpu.make_async_copy(gates_hbm.at[pl.ds(gbase(k), E_BLK)], g[b],
                              lsem[b]).wait()

    def scat_start(slot, b):
        pltpu.async_copy(rows[b], acc.at[idx_v.at[4 + slot]], ssem[b],
                         add=True)

    def scat_wait(slot, b):
        pltpu.make_async_copy(rows[b], acc.at[idx_v.at[4 + slot]],
                              ssem[b]).wait()

    def multiply(b):
        rv, gv = rows[b], g[b]

        @plsc.parallel_loop(0, E_BLK, unroll=2)
        def _(e):
            for j in range(CH // 16):
                sl = pl.ds(j * 16, 16)
                rv[e, sl] = rv[e, sl] * gv[e, sl]

    @plsc.parallel_loop(0, E_BLK, unroll=2)
    def _(e):
        for j in range(CH // 16):
            rows0[e, pl.ds(j * 16, 16)] = jnp.zeros((16,), jnp.float32)

    def zacc(k, _):
        pltpu.sync_copy(rows0, acc.at[pl.ds(s * 640 + k * E_BLK, E_BLK)])
        return 0
    lax.fori_loop(0, ACC_ROWS // NS // E_BLK, zacc, 0)
    plsc.subcore_barrier()

    pltpu.sync_copy(src4_hbm.at[c, s, 0], idx_v.at[0])
    pltpu.sync_copy(dst3_hbm.at[s, 0], idx_v.at[4])
    idx_load(1, 1)
    load_start(0, 0, 0)

    def substep(k, q, b4):
        b = b4 % 2
        load_wait(k, b4, b)

        @pl.when(k > 0)
        def _():
            scat_wait((b4 + 3) % 4, 1 - b)

        @pl.when(k < CHUNKS - 2)
        def _():
            idx_load(k + 2, (b4 + 2) % 4)

        idx_wait(k + 1, (b4 + 1) % 4)
        load_start(k + 1, (b4 + 1) % 4, 1 - b)
        multiply(b)
        scat_start(b4, b)

    def quad(q, _):
        for b4 in range(4):
            substep(4 * q + b4, q, b4)
        return 0
    lax.fori_loop(0, (CHUNKS - 1) // 4, quad, 0)

    kt = CHUNKS - 1
    load_wait(kt, kt % 4, kt % 2)
    scat_wait((kt + 3) % 4, 1 - kt % 2)
    multiply(kt % 2)
    scat_start(kt % 4, kt % 2)
    scat_wait(kt % 4, kt % 2)
    plsc.subcore_barrier()

    @pl.when(s < NS - 1)
    def _():
        off = s * 624
        pltpu.sync_copy(acc.at[pl.ds(off, 624)],
                        out_hbm.at[pl.ds(c * N_NODES + off, 624)])

    @pl.when(s == NS - 1)
    def _():
        pltpu.sync_copy(acc.at[pl.ds(9360, 640)],
                        out_hbm.at[pl.ds(c * N_NODES + 9360, 640)])


def _sc_agg(src4, dst3, x2, gates):
    mesh = plsc.VectorSubcoreMesh(core_axis_name="c", subcore_axis_name="s")
    fn = functools.partial(
        pl.kernel,
        mesh=mesh,
        out_type=jax.ShapeDtypeStruct((2 * N_NODES, CH), jnp.float32),
        scratch_types=[
            pltpu.VMEM((8, E_BLK), jnp.int32),
            pltpu.VMEM((E_BLK, CH), jnp.float32),
            pltpu.VMEM((E_BLK, CH), jnp.float32),
            pltpu.VMEM((E_BLK, CH), jnp.float32),
            pltpu.VMEM((E_BLK, CH), jnp.float32),
            pltpu.VMEM_SHARED((ACC_ROWS, CH), jnp.float32),
        ] + [pltpu.SemaphoreType.DMA] * 8,
    )(_sc_agg_body)
    return fn(src4, dst3, x2, gates)


def _combine_body(x_ref, a0_ref, a1_ref, w1_ref, w2a_ref, w2b_ref, b_ref, out_ref):
    acc = jnp.dot(x_ref[...], w1_ref[...], preferred_element_type=jnp.float32)
    acc += jnp.dot(a0_ref[...], w2a_ref[...], preferred_element_type=jnp.float32)
    acc += jnp.dot(a1_ref[...], w2b_ref[...], preferred_element_type=jnp.float32)
    out_ref[...] = acc + b_ref[...]


def _combine_tc(x, agg2, W1t, W2at, W2bt, bc2):
    BM = 1000
    nb = N_NODES // BM
    return pl.pallas_call(
        _combine_body,
        grid=(nb,),
        in_specs=[
            pl.BlockSpec((BM, 256), lambda m: (m, 0)),
            pl.BlockSpec((BM, CH), lambda m: (m, 0)),
            pl.BlockSpec((BM, CH), lambda m: (m + nb, 0)),
            pl.BlockSpec((256, 256), lambda m: (0, 0)),
            pl.BlockSpec((CH, 256), lambda m: (0, 0)),
            pl.BlockSpec((CH, 256), lambda m: (0, 0)),
            pl.BlockSpec((1, 256), lambda m: (0, 0)),
        ],
        out_specs=pl.BlockSpec((BM, 256), lambda m: (m, 0)),
        out_shape=jax.ShapeDtypeStruct((N_NODES, 256), jnp.float32),
    )(x, agg2, agg2, W1t, W2at, W2bt, bc2)


def kernel(x, edge_index, edge_attr, Wg, bg, Wc, bc):
    src = edge_index[0].astype(jnp.int32)
    dst = edge_index[1].astype(jnp.int32)
    src4 = jnp.stack([2 * src, 2 * src + 1]).reshape(2, NS, CHUNKS, E_BLK)
    dst3 = dst.reshape(NS, CHUNKS, E_BLK)
    x2 = x.reshape(2 * N_NODES, CH)

    WgT_s = jnp.stack([Wg[:CH].T, Wg[CH:].T])
    bg_s = jnp.stack([bg[:CH], bg[CH:]]).reshape(2, 1, CH)
    gates = _gates_tc(edge_attr, WgT_s, bg_s)

    agg2 = _sc_agg(src4, dst3, x2, gates)

    WcT = Wc.T
    return _combine_tc(x, agg2, WcT[:256], WcT[256:256 + CH], WcT[256 + CH:],
                       bc.reshape(1, 256))


# speedup vs baseline: 3.3862x; 1.1012x over previous
import functools

import jax
import jax.numpy as jnp
from jax import lax
from jax.experimental import pallas as pl
from jax.experimental.pallas import tpu as pltpu
from jax.experimental.pallas import tpu_sc as plsc

N_NODES = 10000
N_EDGES = 160000
CH = 128
E_BLK = 80
NS = 16
ACC_ROWS = 10240
CHUNKS_A = 66
CHUNKS_B = 59
E_A = CHUNKS_A * E_BLK * NS
E_B = CHUNKS_B * E_BLK * NS


def _gates_body(attr_ref, wT_ref, b_ref, out_ref):
    z = jnp.dot(attr_ref[...], wT_ref[0], preferred_element_type=jnp.float32)
    out_ref[...] = jax.nn.sigmoid(z + b_ref[0])


def _gates_tc(edge_attr, WgT_s, bg_s, bm):
    ne = edge_attr.shape[0]
    nm = ne // bm
    return pl.pallas_call(
        _gates_body,
        grid=(2, nm),
        in_specs=[
            pl.BlockSpec((bm, 16), lambda h, m: (m, 0)),
            pl.BlockSpec((1, 16, CH), lambda h, m: (h, 0, 0)),
            pl.BlockSpec((1, 1, CH), lambda h, m: (h, 0, 0)),
        ],
        out_specs=pl.BlockSpec((bm, CH), lambda h, m: (h * nm + m, 0)),
        out_shape=jax.ShapeDtypeStruct((2 * ne, CH), jnp.float32),
        compiler_params=pltpu.CompilerParams(
            dimension_semantics=("parallel", "arbitrary")),
    )(edge_attr, WgT_s, bg_s)


def _make_sc_body(chunks):
    n_edges_h = chunks * E_BLK * NS
    ept = chunks * E_BLK

    def body(srcf_hbm, dstf_hbm, x2_hbm, gates_hbm, out_hbm,
             idx_v, rows0, rows1, g0, g1, acc,
             lsem0, lsem1, ssem0, ssem1, isem0, isem1, isem2, isem3):
        c = lax.axis_index("c")
        s = lax.axis_index("s")
        rows = (rows0, rows1)
        g = (g0, g1)
        lsem = (lsem0, lsem1)
        ssem = (ssem0, ssem1)
        isem = (isem0, isem1, isem2, isem3)

        def soff(k):
            return c * n_edges_h + s * ept + k * E_BLK

        def doff(k):
            return s * ept + k * E_BLK

        def idx_load(k, slot):
            pltpu.async_copy(srcf_hbm.at[pl.ds(soff(k), E_BLK)],
                             idx_v.at[slot], isem[slot])
            pltpu.async_copy(dstf_hbm.at[pl.ds(doff(k), E_BLK)],
                             idx_v.at[4 + slot], isem[slot])

        def idx_wait(k, slot):
            pltpu.make_async_copy(srcf_hbm.at[pl.ds(soff(k), E_BLK)],
                                  idx_v.at[slot], isem[slot]).wait()
            pltpu.make_async_copy(dstf_hbm.at[pl.ds(doff(k), E_BLK)],
                                  idx_v.at[4 + slot], isem[slot]).wait()

        def gbase(k):
            return c * n_edges_h + s * ept + k * E_BLK

        def load_start(k, slot, b):
            pltpu.async_copy(x2_hbm.at[idx_v.at[slot]], rows[b], lsem[b])
            pltpu.async_copy(gates_hbm.at[pl.ds(gbase(k), E_BLK)], g[b],
                             lsem[b])

        def load_wait(k, slot, b):
            pltpu.make_async_copy(x2_hbm.at[idx_v.at[slot]], rows[b],
                                  lsem[b]).wait()
            pltpu.make_async_copy(gates_hbm.at[pl.ds(gbase(k), E_BLK)], g[b],
                                  lsem[b]).wait()

        def scat_start(slot, b):
            pltpu.async_copy(rows[b], acc.at[idx_v.at[4 + slot]], ssem[b],
                             add=True)

        def scat_wait(slot, b):
            pltpu.make_async_copy(rows[b], acc.at[idx_v.at[4 + slot]],
                                  ssem[b]).wait()

        def multiply(b):
            rv, gv = rows[b], g[b]

            @plsc.parallel_loop(0, E_BLK, unroll=2)
            def _(e):
                for j in range(CH // 16):
                    sl = pl.ds(j * 16, 16)
                    rv[e, sl] = rv[e, sl] * gv[e, sl]

        @plsc.parallel_loop(0, E_BLK, unroll=2)
        def _(e):
            for j in range(CH // 16):
                rows0[e, pl.ds(j * 16, 16)] = jnp.zeros((16,), jnp.float32)

        def zacc(k, _):
            pltpu.sync_copy(rows0, acc.at[pl.ds(s * 640 + k * E_BLK, E_BLK)])
            return 0
        lax.fori_loop(0, ACC_ROWS // NS // E_BLK, zacc, 0)
        plsc.subcore_barrier()

        pltpu.sync_copy(srcf_hbm.at[pl.ds(soff(0), E_BLK)], idx_v.at[0])
        pltpu.sync_copy(dstf_hbm.at[pl.ds(doff(0), E_BLK)], idx_v.at[4])
        idx_load(1, 1)
        load_start(0, 0, 0)

        def substep(k, b4):
            b = b4 % 2
            load_wait(k, b4, b)

            @pl.when(k > 0)
            def _():
                scat_wait((b4 + 3) % 4, 1 - b)

            @pl.when(k < chunks - 2)
            def _():
                idx_load(k + 2, (b4 + 2) % 4)

            idx_wait(k + 1, (b4 + 1) % 4)
            load_start(k + 1, (b4 + 1) % 4, 1 - b)
            multiply(b)
            scat_start(b4, b)

        nq = (chunks - 1) // 4
        rem = (chunks - 1) - 4 * nq

        def quad(q, _):
            for b4 in range(4):
                substep(4 * q + b4, b4)
            return 0
        lax.fori_loop(0, nq, quad, 0)
        for b4 in range(rem):
            substep(4 * nq + b4, b4)

        kt = chunks - 1
        load_wait(kt, kt % 4, kt % 2)
        scat_wait((kt + 3) % 4, 1 - kt % 2)
        multiply(kt % 2)
        scat_start(kt % 4, kt % 2)
        scat_wait(kt % 4, kt % 2)
        plsc.subcore_barrier()

        @pl.when(s < NS - 1)
        def _():
            off = s * 624
            pltpu.sync_copy(acc.at[pl.ds(off, 624)],
                            out_hbm.at[pl.ds(c * N_NODES + off, 624)])

        @pl.when(s == NS - 1)
        def _():
            pltpu.sync_copy(acc.at[pl.ds(9360, 640)],
                            out_hbm.at[pl.ds(c * N_NODES + 9360, 640)])

    return body


def _sc_agg(srcf, dstf, x2, gates, chunks):
    mesh = plsc.VectorSubcoreMesh(core_axis_name="c", subcore_axis_name="s")
    fn = functools.partial(
        pl.kernel,
        mesh=mesh,
        out_type=jax.ShapeDtypeStruct((2 * N_NODES, CH), jnp.float32),
        scratch_types=[
            pltpu.VMEM((8, E_BLK), jnp.int32),
            pltpu.VMEM((E_BLK, CH), jnp.float32),
            pltpu.VMEM((E_BLK, CH), jnp.float32),
            pltpu.VMEM((E_BLK, CH), jnp.float32),
            pltpu.VMEM((E_BLK, CH), jnp.float32),
            pltpu.VMEM_SHARED((ACC_ROWS, CH), jnp.float32),
        ] + [pltpu.SemaphoreType.DMA] * 8,
    )(_make_sc_body(chunks))
    return fn(srcf, dstf, x2, gates)


def _combine_body(x_ref, a0a_ref, a1a_ref, a0b_ref, a1b_ref,
                  w1_ref, w2a_ref, w2b_ref, b_ref, out_ref):
    acc = jnp.dot(x_ref[...], w1_ref[...], preferred_element_type=jnp.float32)
    acc += jnp.dot(a0a_ref[...] + a0b_ref[...], w2a_ref[...],
                   preferred_element_type=jnp.float32)
    acc += jnp.dot(a1a_ref[...] + a1b_ref[...], w2b_ref[...],
                   preferred_element_type=jnp.float32)
    out_ref[...] = acc + b_ref[...]


def _combine_tc(x, agg_a, agg_b, W1t, W2at, W2bt, bc2):
    BM = 1000
    nb = N_NODES // BM
    return pl.pallas_call(
        _combine_body,
        grid=(nb,),
        in_specs=[
            pl.BlockSpec((BM, 256), lambda m: (m, 0)),
            pl.BlockSpec((BM, CH), lambda m: (m, 0)),
            pl.BlockSpec((BM, CH), lambda m: (m + nb, 0)),
            pl.BlockSpec((BM, CH), lambda m: (m, 0)),
            pl.BlockSpec((BM, CH), lambda m: (m + nb, 0)),
            pl.BlockSpec((256, 256), lambda m: (0, 0)),
            pl.BlockSpec((CH, 256), lambda m: (0, 0)),
            pl.BlockSpec((CH, 256), lambda m: (0, 0)),
            pl.BlockSpec((1, 256), lambda m: (0, 0)),
        ],
        out_specs=pl.BlockSpec((BM, 256), lambda m: (m, 0)),
        out_shape=jax.ShapeDtypeStruct((N_NODES, 256), jnp.float32),
    )(x, agg_a, agg_a, agg_b, agg_b, W1t, W2at, W2bt, bc2)


def _edge_prep(src_h, dst_h):
    srcf = jnp.stack([2 * src_h, 2 * src_h + 1]).reshape(-1)
    return srcf, dst_h


def kernel(x, edge_index, edge_attr, Wg, bg, Wc, bc):
    src = edge_index[0].astype(jnp.int32)
    dst = edge_index[1].astype(jnp.int32)
    x2 = x.reshape(2 * N_NODES, CH)

    WgT_s = jnp.stack([Wg[:CH].T, Wg[CH:].T])
    bg_s = jnp.stack([bg[:CH], bg[CH:]]).reshape(2, 1, CH)

    srcf_a, dstf_a = _edge_prep(src[:E_A], dst[:E_A])
    srcf_b, dstf_b = _edge_prep(src[E_A:], dst[E_A:])

    gates_a = _gates_tc(edge_attr[:E_A], WgT_s, bg_s, 5280)
    gates_b = _gates_tc(edge_attr[E_A:], WgT_s, bg_s, 4720)

    agg_a = _sc_agg(srcf_a, dstf_a, x2, gates_a, CHUNKS_A)
    agg_b = _sc_agg(srcf_b, dstf_b, x2, gates_b, CHUNKS_B)

    WcT = Wc.T
    return _combine_tc(x, agg_a, agg_b, WcT[:256], WcT[256:256 + CH],
                       WcT[256 + CH:], bc.reshape(1, 256))
